# baseline (speedup 1.0000x reference)
import functools

import jax
import jax.numpy as jnp
from jax import lax
from jax.experimental import pallas as pl
from jax.experimental.pallas import tpu as pltpu
from jax.experimental.pallas import tpu_sc as plsc

_K = 4
_STD = 5.0
_BLK = 2048
_NW = 32
_NSPLIT = 2
_CHUNK = 64
_NG = 128


def _topk_body(lrow_ref, labt_ref, lcol_ref, i0_ref, i1_ref, i2_ref, i3_ref,
               w0_ref, w1_ref, w2_ref, w3_ref):
    B = lrow_ref.shape[1]
    R = lcol_ref.shape[0]
    NA = B // _NG

    lab_col = lcol_ref[:, :]

    gm = None
    for a in range(NA):
        d = jnp.abs(lrow_ref[:, a * _NG:(a + 1) * _NG] - lab_col)
        gm = d if gm is None else jnp.minimum(gm, d)

    iota_b = lax.broadcasted_iota(jnp.int32, (R, _NG), 1)
    bts = []
    for t in range(_K):
        if t == 0:
            m = jnp.zeros((R, 1), jnp.float32)
        else:
            m = jnp.min(gm, axis=1, keepdims=True)
        bt = jnp.min(jnp.where(gm == m, iota_b, _NG), axis=1, keepdims=True)
        bts.append(bt)
        if t < _K - 1:
            gm = jnp.where(iota_b == bt, jnp.inf, gm)

    iota_a = lax.broadcasted_iota(jnp.int32, (R, NA), 1)
    cands = []
    colidxs = []
    for t in range(_K):
        oh = jnp.where(iota_b == bts[t], 1.0, 0.0)
        glab = jnp.dot(oh, labt_ref[:, :], precision=lax.Precision.HIGHEST,
                       preferred_element_type=jnp.float32)
        cands.append(jnp.abs(glab - lab_col))
        colidxs.append(iota_a * _NG + bts[t])

    inv2s2 = 1.0 / (2.0 * _STD * _STD)
    big = jnp.int32(B)
    idxs = []
    ws = []
    for k in range(_K):
        mm = jnp.minimum(jnp.minimum(cands[0], cands[1]),
                         jnp.minimum(cands[2], cands[3]))
        m = jnp.min(mm, axis=1, keepdims=True)
        c2 = [jnp.where(cands[t] == m, colidxs[t], big) for t in range(_K)]
        jj = jnp.minimum(jnp.minimum(c2[0], c2[1]),
                         jnp.minimum(c2[2], c2[3]))
        jdx = jnp.min(jj, axis=1, keepdims=True)
        idxs.append(jdx)
        ws.append(jnp.exp(-(m * m) * inv2s2))
        if k < _K - 1:
            for t in range(_K):
                cands[t] = jnp.where(colidxs[t] == jdx, jnp.inf, cands[t])

    rs = 1.0 / (ws[0] + ws[1] + ws[2] + ws[3])
    for k, (iref, wref) in enumerate(
            zip((i0_ref, i1_ref, i2_ref, i3_ref),
                (w0_ref, w1_ref, w2_ref, w3_ref))):
        iref[:, :] = idxs[k]
        wref[:, :] = ws[k] * rs


def _sc_gather_body(struct_hbm, i0, i1, i2, i3, g0, g1, g2, g3,
                    idx_v, rows_v, s0, s1, s2, s3):
    wid = lax.axis_index("s") * 2 + lax.axis_index("c")
    rows_per_w = i0.shape[0] // _NW
    base = wid * rows_per_w
    sems = (s0, s1, s2, s3)
    idx_hbms = (i0, i1, i2, i3)
    g_hbms = (g0, g1, g2, g3)
    for c in range(rows_per_w // _CHUNK):
        rb = base + c * _CHUNK
        copies = []
        for k in range(_K):
            pltpu.sync_copy(idx_hbms[k].at[pl.ds(rb, _CHUNK)], idx_v.at[k])
            copies.append(
                pltpu.async_copy(struct_hbm.at[idx_v.at[k]], rows_v.at[k],
                                 sems[k]))
        for k in range(_K):
            copies[k].wait()
            pltpu.sync_copy(rows_v.at[k], g_hbms[k].at[pl.ds(rb, _CHUNK)])


def _combine_body(s_ref, g0_ref, g1_ref, g2_ref, g3_ref,
                  w0_ref, w1_ref, w2_ref, w3_ref, prev_ref, out_ref,
                  *, finalize, total_b):
    i = pl.program_id(0)

    mean = (w0_ref[:, :] * g0_ref[:, :] + w1_ref[:, :] * g1_ref[:, :]
            + w2_ref[:, :] * g2_ref[:, :] + w3_ref[:, :] * g3_ref[:, :])
    s = s_ref[:, :]
    n1 = jnp.sqrt(jnp.sum(s * s, axis=1, keepdims=True))
    n2 = jnp.sqrt(jnp.sum(mean * mean, axis=1, keepdims=True))
    sm = (s / (1e-10 + n1)) * (mean / (1e-10 + n2))
    partial = jnp.sum(sm)

    @pl.when(i == 0)
    def _():
        out_ref[0, 0] = prev_ref[0, 0]

    out_ref[0, 0] += partial

    if finalize:
        @pl.when(i == pl.num_programs(0) - 1)
        def _():
            out_ref[0, 0] = 1.0 - out_ref[0, 0] / total_b


def kernel(Struct, Label):
    B, D = Struct.shape
    H = B // _NSPLIT
    NA = B // _NG
    lrow = Label.reshape(1, B)
    lcol = Label.reshape(B, 1)
    labt = Label.reshape(NA, _NG).T
    grid_h = H // _BLK

    col_i = jax.ShapeDtypeStruct((H, 1), jnp.int32)
    col_f = jax.ShapeDtypeStruct((H, 1), jnp.float32)
    topk_half = pl.pallas_call(
        _topk_body,
        grid=(grid_h,),
        in_specs=[
            pl.BlockSpec((1, B), lambda i: (0, 0)),
            pl.BlockSpec((_NG, NA), lambda i: (0, 0)),
            pl.BlockSpec((_BLK, 1), lambda i: (i, 0)),
        ],
        out_specs=[pl.BlockSpec((_BLK, 1), lambda i: (i, 0))] * 8,
        out_shape=[col_i] * 4 + [col_f] * 4,
    )

    mesh = plsc.VectorSubcoreMesh(core_axis_name="c", subcore_axis_name="s")
    rows_per_w = H // _NW
    gtype = jax.ShapeDtypeStruct((H, D), jnp.float32)
    sc_gather = functools.partial(
        pl.kernel, mesh=mesh,
        out_type=(gtype,) * 4,
        scratch_types=[
            pltpu.VMEM((_K, _CHUNK), jnp.int32),
            pltpu.VMEM((_K, _CHUNK, D), jnp.float32),
            pltpu.SemaphoreType.DMA,
            pltpu.SemaphoreType.DMA,
            pltpu.SemaphoreType.DMA,
            pltpu.SemaphoreType.DMA,
        ],
    )(_sc_gather_body)

    acc = jnp.zeros((1, 1), jnp.float32)
    for h in range(_NSPLIT):
        i0, i1, i2, i3, w0, w1, w2, w3 = topk_half(
            lrow, labt, lax.slice(lcol, (h * H, 0), ((h + 1) * H, 1)))
        g0, g1, g2, g3 = sc_gather(
            Struct, i0.reshape(H), i1.reshape(H), i2.reshape(H),
            i3.reshape(H))
        acc = pl.pallas_call(
            functools.partial(_combine_body,
                              finalize=(h == _NSPLIT - 1), total_b=B),
            grid=(grid_h,),
            in_specs=[pl.BlockSpec((_BLK, D), lambda i, h=h: (h * grid_h + i, 0))]
            + [pl.BlockSpec((_BLK, D), lambda i: (i, 0))] * 4
            + [pl.BlockSpec((_BLK, 1), lambda i: (i, 0))] * 4
            + [pl.BlockSpec(memory_space=pltpu.SMEM)],
            out_specs=pl.BlockSpec(memory_space=pltpu.SMEM),
            out_shape=jax.ShapeDtypeStruct((1, 1), jnp.float32),
        )(Struct, g0, g1, g2, g3, w0, w1, w2, w3, acc)
    return acc[0, 0]


# speedup vs baseline: 1.1052x; 1.1052x over previous
import functools

import jax
import jax.numpy as jnp
from jax import lax
from jax.experimental import pallas as pl
from jax.experimental.pallas import tpu as pltpu
from jax.experimental.pallas import tpu_sc as plsc

_K = 4
_STD = 5.0
_BLK = 1024
_NW = 32
_NSPLIT = 1
_CHUNK = 64
_NG = 128


def _topk_body(lrow_ref, labt_ref, lcol_ref, i0_ref, i1_ref, i2_ref, i3_ref,
               w0_ref, w1_ref, w2_ref, w3_ref):
    B = lrow_ref.shape[1]
    R = lcol_ref.shape[0]
    NA = B // _NG

    lab_col = lcol_ref[:, :]

    gm = None
    for a in range(NA):
        d = jnp.abs(lrow_ref[:, a * _NG:(a + 1) * _NG] - lab_col)
        gm = d if gm is None else jnp.minimum(gm, d)

    iota_b = lax.broadcasted_iota(jnp.int32, (R, _NG), 1)
    bts = []
    for t in range(_K):
        if t == 0:
            m = jnp.zeros((R, 1), jnp.float32)
        else:
            m = jnp.min(gm, axis=1, keepdims=True)
        bt = jnp.min(jnp.where(gm == m, iota_b, _NG), axis=1, keepdims=True)
        bts.append(bt)
        if t < _K - 1:
            gm = jnp.where(iota_b == bt, jnp.inf, gm)

    iota_a = lax.broadcasted_iota(jnp.int32, (R, NA), 1)
    cands = []
    colidxs = []
    for t in range(_K):
        oh = jnp.where(iota_b == bts[t], 1.0, 0.0)
        glab = jnp.dot(oh, labt_ref[:, :], precision=lax.Precision.HIGHEST,
                       preferred_element_type=jnp.float32)
        cands.append(jnp.abs(glab - lab_col))
        colidxs.append(iota_a * _NG + bts[t])

    inv2s2 = 1.0 / (2.0 * _STD * _STD)
    big = jnp.int32(B)
    idxs = []
    ws = []
    for k in range(_K):
        mm = jnp.minimum(jnp.minimum(cands[0], cands[1]),
                         jnp.minimum(cands[2], cands[3]))
        m = jnp.min(mm, axis=1, keepdims=True)
        c2 = [jnp.where(cands[t] == m, colidxs[t], big) for t in range(_K)]
        jj = jnp.minimum(jnp.minimum(c2[0], c2[1]),
                         jnp.minimum(c2[2], c2[3]))
        jdx = jnp.min(jj, axis=1, keepdims=True)
        idxs.append(jdx)
        ws.append(jnp.exp(-(m * m) * inv2s2))
        if k < _K - 1:
            for t in range(_K):
                cands[t] = jnp.where(colidxs[t] == jdx, jnp.inf, cands[t])

    rs = 1.0 / (ws[0] + ws[1] + ws[2] + ws[3])
    for k, (iref, wref) in enumerate(
            zip((i0_ref, i1_ref, i2_ref, i3_ref),
                (w0_ref, w1_ref, w2_ref, w3_ref))):
        iref[:, :] = idxs[k]
        wref[:, :] = ws[k] * rs


def _sc_gather_body(struct_hbm, i0, i1, i2, i3, g0, g1, g2, g3,
                    idx_v, rows_v, s0, s1, s2, s3):
    wid = lax.axis_index("s") * 2 + lax.axis_index("c")
    rows_per_w = i0.shape[0] // _NW
    base = wid * rows_per_w
    sems = (s0, s1, s2, s3)
    idx_hbms = (i0, i1, i2, i3)
    g_hbms = (g0, g1, g2, g3)
    for c in range(rows_per_w // _CHUNK):
        rb = base + c * _CHUNK
        copies = []
        for k in range(_K):
            pltpu.sync_copy(idx_hbms[k].at[pl.ds(rb, _CHUNK)], idx_v.at[k])
            copies.append(
                pltpu.async_copy(struct_hbm.at[idx_v.at[k]], rows_v.at[k],
                                 sems[k]))
        for k in range(_K):
            copies[k].wait()
            pltpu.sync_copy(rows_v.at[k], g_hbms[k].at[pl.ds(rb, _CHUNK)])


def _combine_body(s_ref, g0_ref, g1_ref, g2_ref, g3_ref,
                  w0_ref, w1_ref, w2_ref, w3_ref, prev_ref, out_ref,
                  *, finalize, total_b):
    i = pl.program_id(0)

    mean = (w0_ref[:, :] * g0_ref[:, :] + w1_ref[:, :] * g1_ref[:, :]
            + w2_ref[:, :] * g2_ref[:, :] + w3_ref[:, :] * g3_ref[:, :])
    s = s_ref[:, :]
    n1 = jnp.sqrt(jnp.sum(s * s, axis=1, keepdims=True))
    n2 = jnp.sqrt(jnp.sum(mean * mean, axis=1, keepdims=True))
    sm = (s / (1e-10 + n1)) * (mean / (1e-10 + n2))
    partial = jnp.sum(sm)

    @pl.when(i == 0)
    def _():
        out_ref[0, 0] = prev_ref[0, 0]

    out_ref[0, 0] += partial

    if finalize:
        @pl.when(i == pl.num_programs(0) - 1)
        def _():
            out_ref[0, 0] = 1.0 - out_ref[0, 0] / total_b


def kernel(Struct, Label):
    B, D = Struct.shape
    H = B // _NSPLIT
    NA = B // _NG
    lrow = Label.reshape(1, B)
    lcol = Label.reshape(B, 1)
    labt = Label.reshape(NA, _NG).T
    grid_h = H // _BLK

    col_i = jax.ShapeDtypeStruct((H, 1), jnp.int32)
    col_f = jax.ShapeDtypeStruct((H, 1), jnp.float32)
    topk_half = pl.pallas_call(
        _topk_body,
        grid=(grid_h,),
        in_specs=[
            pl.BlockSpec((1, B), lambda i: (0, 0)),
            pl.BlockSpec((_NG, NA), lambda i: (0, 0)),
            pl.BlockSpec((_BLK, 1), lambda i: (i, 0)),
        ],
        out_specs=[pl.BlockSpec((_BLK, 1), lambda i: (i, 0))] * 8,
        out_shape=[col_i] * 4 + [col_f] * 4,
    )

    mesh = plsc.VectorSubcoreMesh(core_axis_name="c", subcore_axis_name="s")
    rows_per_w = H // _NW
    gtype = jax.ShapeDtypeStruct((H, D), jnp.float32)
    sc_gather = functools.partial(
        pl.kernel, mesh=mesh,
        out_type=(gtype,) * 4,
        scratch_types=[
            pltpu.VMEM((_K, _CHUNK), jnp.int32),
            pltpu.VMEM((_K, _CHUNK, D), jnp.float32),
            pltpu.SemaphoreType.DMA,
            pltpu.SemaphoreType.DMA,
            pltpu.SemaphoreType.DMA,
            pltpu.SemaphoreType.DMA,
        ],
    )(_sc_gather_body)

    acc = jnp.zeros((1, 1), jnp.float32)
    for h in range(_NSPLIT):
        i0, i1, i2, i3, w0, w1, w2, w3 = topk_half(
            lrow, labt, lax.slice(lcol, (h * H, 0), ((h + 1) * H, 1)))
        g0, g1, g2, g3 = sc_gather(
            Struct, i0.reshape(H), i1.reshape(H), i2.reshape(H),
            i3.reshape(H))
        acc = pl.pallas_call(
            functools.partial(_combine_body,
                              finalize=(h == _NSPLIT - 1), total_b=B),
            grid=(grid_h,),
            in_specs=[pl.BlockSpec((_BLK, D), lambda i, h=h: (h * grid_h + i, 0))]
            + [pl.BlockSpec((_BLK, D), lambda i: (i, 0))] * 4
            + [pl.BlockSpec((_BLK, 1), lambda i: (i, 0))] * 4
            + [pl.BlockSpec(memory_space=pltpu.SMEM)],
            out_specs=pl.BlockSpec(memory_space=pltpu.SMEM),
            out_shape=jax.ShapeDtypeStruct((1, 1), jnp.float32),
        )(Struct, g0, g1, g2, g3, w0, w1, w2, w3, acc)
    return acc[0, 0]


# speedup vs baseline: 1.2481x; 1.1293x over previous
import functools

import jax
import jax.numpy as jnp
from jax import lax
from jax.experimental import pallas as pl
from jax.experimental.pallas import tpu as pltpu
from jax.experimental.pallas import tpu_sc as plsc

_K = 4
_STD = 5.0
_BLK = 1024
_NW = 32
_NSPLIT = 1
_CHUNK = 64
_NG = 128


def _topk_body(lrow_ref, labt_ref, lcol_ref, i1_ref, i2_ref, i3_ref,
               w0_ref, w1_ref, w2_ref, w3_ref):
    B = lrow_ref.shape[1]
    R = lcol_ref.shape[0]
    NA = B // _NG
    row_g = (pl.program_id(0) * R
             + lax.broadcasted_iota(jnp.int32, (R, 1), 0))

    lab_col = lcol_ref[:, :]

    gm = None
    for a in range(NA):
        d = jnp.abs(lrow_ref[:, a * _NG:(a + 1) * _NG] - lab_col)
        gm = d if gm is None else jnp.minimum(gm, d)

    iota_b = lax.broadcasted_iota(jnp.int32, (R, _NG), 1)
    bts = []
    for t in range(_K):
        if t == 0:
            m = jnp.zeros((R, 1), jnp.float32)
        else:
            m = jnp.min(gm, axis=1, keepdims=True)
        bt = jnp.min(jnp.where(gm == m, iota_b, _NG), axis=1, keepdims=True)
        bts.append(bt)
        if t < _K - 1:
            gm = jnp.where(iota_b == bt, jnp.inf, gm)

    iota_a = lax.broadcasted_iota(jnp.int32, (R, NA), 1)
    cands = []
    colidxs = []
    for t in range(_K):
        oh = jnp.where(iota_b == bts[t], 1.0, 0.0)
        glab = jnp.dot(oh, labt_ref[:, :], precision=lax.Precision.HIGHEST,
                       preferred_element_type=jnp.float32)
        colidx = iota_a * _NG + bts[t]
        cands.append(jnp.where(colidx == row_g, jnp.inf,
                               jnp.abs(glab - lab_col)))
        colidxs.append(colidx)

    inv2s2 = 1.0 / (2.0 * _STD * _STD)
    big = jnp.int32(B)
    idxs = []
    ws = [jnp.ones((R, 1), jnp.float32)]
    for k in range(_K - 1):
        mm = jnp.minimum(jnp.minimum(cands[0], cands[1]),
                         jnp.minimum(cands[2], cands[3]))
        m = jnp.min(mm, axis=1, keepdims=True)
        c2 = [jnp.where(cands[t] == m, colidxs[t], big) for t in range(_K)]
        jj = jnp.minimum(jnp.minimum(c2[0], c2[1]),
                         jnp.minimum(c2[2], c2[3]))
        jdx = jnp.min(jj, axis=1, keepdims=True)
        idxs.append(jdx)
        ws.append(jnp.exp(-(m * m) * inv2s2))
        if k < _K - 2:
            for t in range(_K):
                cands[t] = jnp.where(colidxs[t] == jdx, jnp.inf, cands[t])

    rs = 1.0 / (ws[0] + ws[1] + ws[2] + ws[3])
    for iref, jdx in zip((i1_ref, i2_ref, i3_ref), idxs):
        iref[:, :] = jdx
    for wref, w in zip((w0_ref, w1_ref, w2_ref, w3_ref), ws):
        wref[:, :] = w * rs


def _sc_gather_body(struct_hbm, i1, i2, i3, g1, g2, g3,
                    idx_v, rows_v, s0, s1, s2, s3, s4, s5):
    wid = lax.axis_index("s") * 2 + lax.axis_index("c")
    rows_per_w = i1.shape[0] // _NW
    base = wid * rows_per_w
    nc = rows_per_w // _CHUNK
    sems = (s0, s1, s2, s3, s4, s5)
    idx_hbms = (i1, i2, i3)
    g_hbms = (g1, g2, g3)
    copies = []
    for c in range(nc):
        rb = base + c * _CHUNK
        for k in range(3):
            pltpu.sync_copy(idx_hbms[k].at[pl.ds(rb, _CHUNK)],
                            idx_v.at[c, k])
            copies.append(
                pltpu.async_copy(struct_hbm.at[idx_v.at[c, k]],
                                 rows_v.at[c, k], sems[c * 3 + k]))
    for c in range(nc):
        rb = base + c * _CHUNK
        for k in range(3):
            copies[c * 3 + k].wait()
            pltpu.sync_copy(rows_v.at[c, k], g_hbms[k].at[pl.ds(rb, _CHUNK)])


def _combine_body(s_ref, g1_ref, g2_ref, g3_ref,
                  w0_ref, w1_ref, w2_ref, w3_ref, prev_ref, out_ref,
                  *, finalize, total_b):
    i = pl.program_id(0)

    s = s_ref[:, :]
    mean = (w0_ref[:, :] * s + w1_ref[:, :] * g1_ref[:, :]
            + w2_ref[:, :] * g2_ref[:, :] + w3_ref[:, :] * g3_ref[:, :])
    n1 = jnp.sqrt(jnp.sum(s * s, axis=1, keepdims=True))
    n2 = jnp.sqrt(jnp.sum(mean * mean, axis=1, keepdims=True))
    sm = (s / (1e-10 + n1)) * (mean / (1e-10 + n2))
    partial = jnp.sum(sm)

    @pl.when(i == 0)
    def _():
        out_ref[0, 0] = prev_ref[0, 0]

    out_ref[0, 0] += partial

    if finalize:
        @pl.when(i == pl.num_programs(0) - 1)
        def _():
            out_ref[0, 0] = 1.0 - out_ref[0, 0] / total_b


def kernel(Struct, Label):
    B, D = Struct.shape
    H = B // _NSPLIT
    NA = B // _NG
    lrow = Label.reshape(1, B)
    lcol = Label.reshape(B, 1)
    labt = Label.reshape(NA, _NG).T
    grid_h = H // _BLK

    col_i = jax.ShapeDtypeStruct((H, 1), jnp.int32)
    col_f = jax.ShapeDtypeStruct((H, 1), jnp.float32)
    topk_half = pl.pallas_call(
        _topk_body,
        grid=(grid_h,),
        in_specs=[
            pl.BlockSpec((1, B), lambda i: (0, 0)),
            pl.BlockSpec((_NG, NA), lambda i: (0, 0)),
            pl.BlockSpec((_BLK, 1), lambda i: (i, 0)),
        ],
        out_specs=[pl.BlockSpec((_BLK, 1), lambda i: (i, 0))] * 7,
        out_shape=[col_i] * 3 + [col_f] * 4,
    )

    mesh = plsc.VectorSubcoreMesh(core_axis_name="c", subcore_axis_name="s")
    rows_per_w = H // _NW
    nc = rows_per_w // _CHUNK
    gtype = jax.ShapeDtypeStruct((H, D), jnp.float32)
    sc_gather = functools.partial(
        pl.kernel, mesh=mesh,
        out_type=(gtype,) * 3,
        scratch_types=[
            pltpu.VMEM((nc, 3, _CHUNK), jnp.int32),
            pltpu.VMEM((nc, 3, _CHUNK, D), jnp.float32),
        ] + [pltpu.SemaphoreType.DMA] * 6,
    )(_sc_gather_body)

    acc = jnp.zeros((1, 1), jnp.float32)
    for h in range(_NSPLIT):
        i1, i2, i3, w0, w1, w2, w3 = topk_half(
            lrow, labt, lax.slice(lcol, (h * H, 0), ((h + 1) * H, 1)))
        g1, g2, g3 = sc_gather(
            Struct, i1.reshape(H), i2.reshape(H), i3.reshape(H))
        acc = pl.pallas_call(
            functools.partial(_combine_body,
                              finalize=(h == _NSPLIT - 1), total_b=B),
            grid=(grid_h,),
            in_specs=[pl.BlockSpec((_BLK, D), lambda i, h=h: (h * grid_h + i, 0))]
            + [pl.BlockSpec((_BLK, D), lambda i: (i, 0))] * 3
            + [pl.BlockSpec((_BLK, 1), lambda i: (i, 0))] * 4
            + [pl.BlockSpec(memory_space=pltpu.SMEM)],
            out_specs=pl.BlockSpec(memory_space=pltpu.SMEM),
            out_shape=jax.ShapeDtypeStruct((1, 1), jnp.float32),
        )(Struct, g1, g2, g3, w0, w1, w2, w3, acc)
    return acc[0, 0]


# speedup vs baseline: 1.2588x; 1.0085x over previous
import functools

import jax
import jax.numpy as jnp
from jax import lax
from jax.experimental import pallas as pl
from jax.experimental.pallas import tpu as pltpu
from jax.experimental.pallas import tpu_sc as plsc

_K = 4
_STD = 5.0
_BLK = 1024
_NW = 32
_NSPLIT = 1
_CHUNK = 64
_NG = 128


def _topk_body(lrow_ref, labt_ref, lcol_ref, i1_ref, i2_ref, i3_ref,
               w0_ref, w1_ref, w2_ref, w3_ref):
    B = lrow_ref.shape[1]
    R = lcol_ref.shape[0]
    NA = B // _NG
    row_g = (pl.program_id(0) * R
             + lax.broadcasted_iota(jnp.int32, (R, 1), 0))

    lab_col = lcol_ref[:, :]

    gm = None
    for a in range(NA):
        d = jnp.abs(lrow_ref[:, a * _NG:(a + 1) * _NG] - lab_col)
        gm = d if gm is None else jnp.minimum(gm, d)

    iota_b = lax.broadcasted_iota(jnp.int32, (R, _NG), 1)
    key = (lax.bitcast_convert_type(gm, jnp.int32) & ~(_NG - 1)) | iota_b
    bts = []
    for t in range(_K):
        kmin = jnp.min(key, axis=1, keepdims=True)
        bts.append(kmin & (_NG - 1))
        if t < _K - 1:
            key = jnp.where(key == kmin, jnp.int32(0x7FFFFFFF), key)

    iota_a = lax.broadcasted_iota(jnp.int32, (R, NA), 1)
    cands = []
    colidxs = []
    for t in range(_K):
        oh = jnp.where(iota_b == bts[t], 1.0, 0.0)
        glab = jnp.dot(oh, labt_ref[:, :], precision=lax.Precision.HIGHEST,
                       preferred_element_type=jnp.float32)
        colidx = iota_a * _NG + bts[t]
        cands.append(jnp.where(colidx == row_g, jnp.inf,
                               jnp.abs(glab - lab_col)))
        colidxs.append(colidx)

    inv2s2 = 1.0 / (2.0 * _STD * _STD)
    big = jnp.int32(B)
    idxs = []
    ws = [jnp.ones((R, 1), jnp.float32)]
    for k in range(_K - 1):
        mm = jnp.minimum(jnp.minimum(cands[0], cands[1]),
                         jnp.minimum(cands[2], cands[3]))
        m = jnp.min(mm, axis=1, keepdims=True)
        c2 = [jnp.where(cands[t] == m, colidxs[t], big) for t in range(_K)]
        jj = jnp.minimum(jnp.minimum(c2[0], c2[1]),
                         jnp.minimum(c2[2], c2[3]))
        jdx = jnp.min(jj, axis=1, keepdims=True)
        idxs.append(jdx)
        ws.append(jnp.exp(-(m * m) * inv2s2))
        if k < _K - 2:
            for t in range(_K):
                cands[t] = jnp.where(colidxs[t] == jdx, jnp.inf, cands[t])

    rs = 1.0 / (ws[0] + ws[1] + ws[2] + ws[3])
    for iref, jdx in zip((i1_ref, i2_ref, i3_ref), idxs):
        iref[:, :] = jdx
    for wref, w in zip((w0_ref, w1_ref, w2_ref, w3_ref), ws):
        wref[:, :] = w * rs


def _sc_gather_body(struct_hbm, i1, i2, i3, g1, g2, g3,
                    idx_v, rows_v, s0, s1, s2, s3, s4, s5):
    wid = lax.axis_index("s") * 2 + lax.axis_index("c")
    rows_per_w = i1.shape[0] // _NW
    base = wid * rows_per_w
    nc = rows_per_w // _CHUNK
    sems = (s0, s1, s2, s3, s4, s5)
    idx_hbms = (i1, i2, i3)
    g_hbms = (g1, g2, g3)
    copies = []
    for c in range(nc):
        rb = base + c * _CHUNK
        for k in range(3):
            pltpu.sync_copy(idx_hbms[k].at[pl.ds(rb, _CHUNK)],
                            idx_v.at[c, k])
            copies.append(
                pltpu.async_copy(struct_hbm.at[idx_v.at[c, k]],
                                 rows_v.at[c, k], sems[c * 3 + k]))
    for c in range(nc):
        rb = base + c * _CHUNK
        for k in range(3):
            copies[c * 3 + k].wait()
            pltpu.sync_copy(rows_v.at[c, k], g_hbms[k].at[pl.ds(rb, _CHUNK)])


def _combine_body(s_ref, g1_ref, g2_ref, g3_ref,
                  w0_ref, w1_ref, w2_ref, w3_ref, prev_ref, out_ref,
                  *, finalize, total_b):
    i = pl.program_id(0)

    s = s_ref[:, :]
    mean = (w0_ref[:, :] * s + w1_ref[:, :] * g1_ref[:, :]
            + w2_ref[:, :] * g2_ref[:, :] + w3_ref[:, :] * g3_ref[:, :])
    n1 = jnp.sqrt(jnp.sum(s * s, axis=1, keepdims=True))
    n2 = jnp.sqrt(jnp.sum(mean * mean, axis=1, keepdims=True))
    sm = (s / (1e-10 + n1)) * (mean / (1e-10 + n2))
    partial = jnp.sum(sm)

    @pl.when(i == 0)
    def _():
        out_ref[0, 0] = prev_ref[0, 0]

    out_ref[0, 0] += partial

    if finalize:
        @pl.when(i == pl.num_programs(0) - 1)
        def _():
            out_ref[0, 0] = 1.0 - out_ref[0, 0] / total_b


def kernel(Struct, Label):
    B, D = Struct.shape
    H = B // _NSPLIT
    NA = B // _NG
    lrow = Label.reshape(1, B)
    lcol = Label.reshape(B, 1)
    labt = Label.reshape(NA, _NG).T
    grid_h = H // _BLK

    col_i = jax.ShapeDtypeStruct((H, 1), jnp.int32)
    col_f = jax.ShapeDtypeStruct((H, 1), jnp.float32)
    topk_half = pl.pallas_call(
        _topk_body,
        grid=(grid_h,),
        in_specs=[
            pl.BlockSpec((1, B), lambda i: (0, 0)),
            pl.BlockSpec((_NG, NA), lambda i: (0, 0)),
            pl.BlockSpec((_BLK, 1), lambda i: (i, 0)),
        ],
        out_specs=[pl.BlockSpec((_BLK, 1), lambda i: (i, 0))] * 7,
        out_shape=[col_i] * 3 + [col_f] * 4,
    )

    mesh = plsc.VectorSubcoreMesh(core_axis_name="c", subcore_axis_name="s")
    rows_per_w = H // _NW
    nc = rows_per_w // _CHUNK
    gtype = jax.ShapeDtypeStruct((H, D), jnp.float32)
    sc_gather = functools.partial(
        pl.kernel, mesh=mesh,
        out_type=(gtype,) * 3,
        scratch_types=[
            pltpu.VMEM((nc, 3, _CHUNK), jnp.int32),
            pltpu.VMEM((nc, 3, _CHUNK, D), jnp.float32),
        ] + [pltpu.SemaphoreType.DMA] * 6,
    )(_sc_gather_body)

    acc = jnp.zeros((1, 1), jnp.float32)
    for h in range(_NSPLIT):
        i1, i2, i3, w0, w1, w2, w3 = topk_half(
            lrow, labt, lax.slice(lcol, (h * H, 0), ((h + 1) * H, 1)))
        g1, g2, g3 = sc_gather(
            Struct, i1.reshape(H), i2.reshape(H), i3.reshape(H))
        acc = pl.pallas_call(
            functools.partial(_combine_body,
                              finalize=(h == _NSPLIT - 1), total_b=B),
            grid=(grid_h,),
            in_specs=[pl.BlockSpec((_BLK, D), lambda i, h=h: (h * grid_h + i, 0))]
            + [pl.BlockSpec((_BLK, D), lambda i: (i, 0))] * 3
            + [pl.BlockSpec((_BLK, 1), lambda i: (i, 0))] * 4
            + [pl.BlockSpec(memory_space=pltpu.SMEM)],
            out_specs=pl.BlockSpec(memory_space=pltpu.SMEM),
            out_shape=jax.ShapeDtypeStruct((1, 1), jnp.float32),
        )(Struct, g1, g2, g3, w0, w1, w2, w3, acc)
    return acc[0, 0]
